# baseline (speedup 1.0000x reference)
import functools

import jax
import jax.numpy as jnp
from jax import lax
from jax.experimental import pallas as pl
from jax.experimental.pallas import tpu as pltpu
from jax.experimental.pallas import tpu_sc as plsc

_B4 = 16384
_F = 64
_W = 128
_NW = 32
_NPW = _B4 // _NW
_CHUNK = 128
_NCH = _NPW // _CHUNK
_SUB = 256
_NGS = _SUB // 16

_mesh = plsc.VectorSubcoreMesh(core_axis_name="c", subcore_axis_name="s")


@functools.partial(
    pl.kernel,
    mesh=_mesh,
    compiler_params=pltpu.CompilerParams(
        needs_layout_passes=False, use_tc_tiling_on_sc=False),
    out_type=jax.ShapeDtypeStruct((_B4,), jnp.float32),
    scratch_types=[
        pltpu.VMEM((_NCH, _CHUNK), jnp.int32),
        pltpu.VMEM((_NCH, _CHUNK), jnp.int32),
        pltpu.VMEM((_NPW,), jnp.int32),
        pltpu.VMEM((_NPW,), jnp.int32),
        pltpu.VMEM((_NPW,), jnp.int32),
        pltpu.VMEM((_SUB, _W), jnp.float32),
        pltpu.VMEM((_SUB, _W), jnp.float32),
        pltpu.VMEM((2 * _F,), jnp.float32),
        pltpu.VMEM((_SUB * 16,), jnp.float32),
        pltpu.VMEM((_NPW,), jnp.float32),
        pltpu.SemaphoreType.DMA,
    ],
)
def _sc_score(hrow_hbm, trow_hbm, hhalf_hbm, thalf_hbm, rel_hbm, table_hbm,
              r_hbm, out_hbm,
              hidx, tidx, hhalf, thalf, ridx, hrows, trows, r_v, pbuf, outv,
              sem):
    wid = lax.axis_index("s") * 2 + lax.axis_index("c")
    base = wid * _NPW

    pltpu.sync_copy(hrow_hbm.at[pl.ds(wid * _NCH, _NCH)], hidx)
    pltpu.sync_copy(trow_hbm.at[pl.ds(wid * _NCH, _NCH)], tidx)
    pltpu.sync_copy(hhalf_hbm.at[pl.ds(base, _NPW)], hhalf)
    pltpu.sync_copy(thalf_hbm.at[pl.ds(base, _NPW)], thalf)
    pltpu.sync_copy(rel_hbm.at[pl.ds(base, _NPW)], ridx)
    pltpu.sync_copy(r_hbm, r_v)

    r0 = [r_v[pl.ds(c * 16, 16)] for c in range(4)]
    r1 = [r_v[pl.ds(_F + c * 16, 16)] for c in range(4)]
    iota16 = lax.iota(jnp.int32, 16)

    for sub in range(2):
        copies = []
        for j in range(2):
            ch = sub * 2 + j
            copies.append(pltpu.async_copy(
                table_hbm.at[hidx.at[ch]],
                hrows.at[pl.ds(j * _CHUNK, _CHUNK)], sem))
            copies.append(pltpu.async_copy(
                table_hbm.at[tidx.at[ch]],
                trows.at[pl.ds(j * _CHUNK, _CHUNK)], sem))
        for cp in copies:
            cp.wait()

        def passa(g, carry):
            sbase = sub * _SUB + g * 16
            relv = ridx[pl.ds(sbase, 16)]
            hhv = hhalf[pl.ds(sbase, 16)]
            thv = thalf[pl.ds(sbase, 16)]
            for k in range(16):
                i = g * 16 + k
                is1 = relv[k] != 0
                ho = hhv[k]
                to = thv[k]
                acc4 = jnp.zeros((16,), jnp.float32)
                for c in range(4):
                    hv = hrows[i, pl.ds(ho + c * 16, 16)]
                    tv = trows[i, pl.ds(to + c * 16, 16)]
                    rv = jnp.where(is1, r1[c], r0[c])
                    s = hv - tv + rv
                    acc4 = acc4 + s * s
                pbuf[pl.ds(i * 16, 16)] = acc4
            return carry

        lax.fori_loop(0, _NGS, passa, 0)

        def passb(g, carry):
            lanebase = g * 256 + iota16 * 16
            acc = jnp.zeros((16,), jnp.float32)
            for j in range(16):
                acc = acc + plsc.load_gather(pbuf, [lanebase + j])
            outv[pl.ds(sub * _SUB + g * 16, 16)] = -acc
            return carry

        lax.fori_loop(0, _NGS, passb, 0)

    pltpu.sync_copy(outv, out_hbm.at[pl.ds(base, _NPW)])


def kernel(item_embedding, r_weight, head_ids, tail_ids, relation_ids):
    table2 = item_embedding.reshape(500000, _W)
    heads = head_ids.reshape(_B4).astype(jnp.int32)
    tails = tail_ids.reshape(_B4).astype(jnp.int32)
    hrow = (heads >> 1).reshape(_NW * _NCH, _CHUNK)
    trow = (tails >> 1).reshape(_NW * _NCH, _CHUNK)
    hhalf = (heads & 1) * _F
    thalf = (tails & 1) * _F
    rels = relation_ids.reshape(_B4).astype(jnp.int32)
    r_flat = r_weight.reshape(2 * _F)
    out = _sc_score(hrow, trow, hhalf, thalf, rels, table2, r_flat)
    return out.reshape(head_ids.shape)
